# Initial kernel scaffold; baseline (speedup 1.0000x reference)
import jax
import jax.numpy as jnp
from jax.experimental import pallas as pl


def kernel(x, edge_index, W1_l, b1_l, W1_r, W2_l, b2_l, W2_r, W_s):
    raise NotImplementedError("write your pallas kernel here")


# speedup vs baseline: 4.1354x; 4.1354x over previous
import functools

import jax
import jax.numpy as jnp
from jax import lax
from jax.experimental import pallas as pl
from jax.experimental.pallas import tpu as pltpu
from jax.experimental.pallas import tpu_sc as plsc

N = 10000
D = 128
D_S = 64
E = 320000

NC = 2
NS = 16
HC = D // NC

B_E = 128
T_STEPS = 160
E_PAD = NS * T_STEPS * B_E
N_PAD = 10240
ROWS_PT = N_PAD // NS
RCH = ROWS_PT // B_E

_mesh = plsc.VectorSubcoreMesh(
    core_axis_name="c", subcore_axis_name="s", num_cores=NC, num_subcores=NS)
_sc_params = pltpu.CompilerParams(use_tc_tiling_on_sc=False)


def _make_sc_agg(with_counts):
  def body(x_hbm, edges_hbm, *refs):
    if with_counts:
      (out_sum, out_cnt, srcv, dstv, buf, zbuf, cbuf, z16, acc, accc,
       sem) = refs
    else:
      out_sum, srcv, dstv, buf, zbuf, acc, sem = refs
      out_cnt = cbuf = z16 = accc = None

    c = lax.axis_index("c")
    s = lax.axis_index("s")

    def initbufs(i, _):
      for t in range(HC // 16):
        zbuf[i, pl.ds(t * 16, 16)] = jnp.zeros((16,), jnp.float32)
      if with_counts:
        cbuf[i, :] = jnp.ones((16,), jnp.float32)
        z16[i, :] = jnp.zeros((16,), jnp.float32)
      return 0

    lax.fori_loop(0, B_E, initbufs, 0)

    for k in range(RCH):
      sl = pl.ds(s * ROWS_PT + k * B_E, B_E)
      pltpu.sync_copy(zbuf, acc.at[sl])
      if with_counts:
        pltpu.sync_copy(z16, accc.at[sl])
    plsc.subcore_barrier()

    pltpu.sync_copy(edges_hbm.at[0, pl.ds(s * T_STEPS, T_STEPS)], srcv)
    pltpu.sync_copy(edges_hbm.at[1, pl.ds(s * T_STEPS, T_STEPS)], dstv)

    xc = x_hbm.at[c]

    def step(j, _):
      pltpu.async_copy(xc.at[srcv.at[j]], buf, sem).wait()
      pltpu.sync_copy(buf, acc.at[dstv.at[j]], add=True)
      if with_counts:
        @pl.when(c == 0)
        def _():
          pltpu.sync_copy(cbuf, accc.at[dstv.at[j]], add=True)
      return 0

    lax.fori_loop(0, T_STEPS, step, 0)
    plsc.subcore_barrier()

    for k in range(RCH):
      sl = pl.ds(s * ROWS_PT + k * B_E, B_E)
      pltpu.sync_copy(acc.at[sl], zbuf)
      pltpu.sync_copy(zbuf, out_sum.at[c, sl])
      if with_counts:
        @pl.when(c == 0)
        def _():
          pltpu.sync_copy(accc.at[sl], cbuf)
          pltpu.sync_copy(cbuf, out_cnt.at[sl])

  out_type = [jax.ShapeDtypeStruct((NC, N_PAD, HC), jnp.float32)]
  scratch = [
      pltpu.VMEM((T_STEPS, B_E), jnp.int32),
      pltpu.VMEM((T_STEPS, B_E), jnp.int32),
      pltpu.VMEM((B_E, HC), jnp.float32),
      pltpu.VMEM((B_E, HC), jnp.float32),
  ]
  if with_counts:
    out_type.append(jax.ShapeDtypeStruct((N_PAD, 16), jnp.float32))
    scratch += [
        pltpu.VMEM((B_E, 16), jnp.float32),
        pltpu.VMEM((B_E, 16), jnp.float32),
    ]
  scratch.append(pltpu.VMEM_SHARED((N_PAD, HC), jnp.float32))
  if with_counts:
    scratch.append(pltpu.VMEM_SHARED((N_PAD, 16), jnp.float32))
  scratch.append(pltpu.SemaphoreType.DMA)

  return pl.kernel(body, out_type=out_type, mesh=_mesh,
                   scratch_types=scratch, compiler_params=_sc_params)


_sc_agg_cnt = _make_sc_agg(True)
_sc_agg = _make_sc_agg(False)

_BN = 1024


def _dense1_body(s_ref, c_ref, x_ref, wl_ref, wr_ref, b_ref, o_ref):
  cnt = c_ref[:, 0:1]
  mean = jnp.concatenate([s_ref[0], s_ref[1]], axis=1) / jnp.maximum(cnt, 1.0)
  h = (jnp.dot(mean, wl_ref[...], preferred_element_type=jnp.float32)
       + b_ref[...]
       + jnp.dot(x_ref[...], wr_ref[...], preferred_element_type=jnp.float32))
  h = jnp.maximum(h, 0.0)
  o_ref[0] = h[:, :HC]
  o_ref[1] = h[:, HC:]


def _dense2_body(s_ref, c_ref, h_ref, wl_ref, wr_ref, b_ref, ws_ref,
                 o_ref, os_ref):
  cnt = c_ref[:, 0:1]
  mean = jnp.concatenate([s_ref[0], s_ref[1]], axis=1) / jnp.maximum(cnt, 1.0)
  h = jnp.concatenate([h_ref[0], h_ref[1]], axis=1)
  h2 = (jnp.dot(mean, wl_ref[...], preferred_element_type=jnp.float32)
        + b_ref[...]
        + jnp.dot(h, wr_ref[...], preferred_element_type=jnp.float32))
  o_ref[...] = h2
  os_ref[...] = jnp.dot(h2, ws_ref[...], preferred_element_type=jnp.float32)


_split_spec = pl.BlockSpec((NC, _BN, HC), lambda i: (0, i, 0))
_cnt_spec = pl.BlockSpec((_BN, 16), lambda i: (i, 0))
_row_spec = pl.BlockSpec((_BN, D), lambda i: (i, 0))
_w_spec = pl.BlockSpec((D, D), lambda i: (0, 0))
_b_spec = pl.BlockSpec((1, D), lambda i: (0, 0))


def _dense1(s1, cnt, xp, wl, wr, b):
  return pl.pallas_call(
      _dense1_body,
      grid=(N_PAD // _BN,),
      in_specs=[_split_spec, _cnt_spec, _row_spec, _w_spec, _w_spec, _b_spec],
      out_specs=_split_spec,
      out_shape=jax.ShapeDtypeStruct((NC, N_PAD, HC), jnp.float32),
  )(s1, cnt, xp, wl, wr, b)


def _dense2(s2, cnt, h, wl, wr, b, ws):
  ws_spec = pl.BlockSpec((D, D_S), lambda i: (0, 0))
  os_spec = pl.BlockSpec((_BN, D_S), lambda i: (i, 0))
  return pl.pallas_call(
      _dense2_body,
      grid=(N_PAD // _BN,),
      in_specs=[_split_spec, _cnt_spec, _split_spec, _w_spec, _w_spec,
                _b_spec, ws_spec],
      out_specs=[_row_spec, os_spec],
      out_shape=[jax.ShapeDtypeStruct((N_PAD, D), jnp.float32),
                 jax.ShapeDtypeStruct((N_PAD, D_S), jnp.float32)],
  )(s2, cnt, h, wl, wr, b, ws)


@jax.jit
def kernel(x, edge_index, W1_l, b1_l, W1_r, W2_l, b2_l, W2_r, W_s):
  xp = jnp.pad(x, ((0, N_PAD - N), (0, 0)))
  xs = xp.reshape(N_PAD, NC, HC).transpose(1, 0, 2)
  src = jnp.pad(edge_index[0], (0, E_PAD - E))
  dst = jnp.pad(edge_index[1], (0, E_PAD - E), constant_values=N)
  edges = jnp.stack([src, dst]).reshape(2, NS * T_STEPS, B_E)

  s1, cnt = _sc_agg_cnt(xs, edges)
  hs = _dense1(s1, cnt, xp, W1_l.T, W1_r.T, b1_l.reshape(1, D))
  (s2,) = _sc_agg(hs, edges)
  h2, out_s = _dense2(s2, cnt, hs, W2_l.T, W2_r.T, b2_l.reshape(1, D),
                      W_s.T)
  return out_s[:N], h2[:N]


# speedup vs baseline: 4.6075x; 1.1142x over previous
import functools

import jax
import jax.numpy as jnp
from jax import lax
from jax.experimental import pallas as pl
from jax.experimental.pallas import tpu as pltpu
from jax.experimental.pallas import tpu_sc as plsc

N = 10000
D = 128
D_S = 64
E = 320000

NC = 2
NS = 16
HC = D // NC

B_E = 128
T_STEPS = 160
E_PAD = NS * T_STEPS * B_E
N_PAD = 10240
ROWS_PT = N_PAD // NS
RCH = ROWS_PT // B_E

RING = 2
DEPTH = RING // 2
NGRP = T_STEPS // RING
CNT_STEPS = T_STEPS // NC

_mesh = plsc.VectorSubcoreMesh(
    core_axis_name="c", subcore_axis_name="s", num_cores=NC, num_subcores=NS)
_sc_params = pltpu.CompilerParams(use_tc_tiling_on_sc=False)


def _make_sc_agg(with_counts):
  def body(x_hbm, edges_hbm, *refs):
    if with_counts:
      out_sum, out_cnt = refs[0], refs[1]
      rest = refs[2:]
    else:
      out_sum = refs[0]
      out_cnt = None
      rest = refs[1:]
    srcv, dstv = rest[0], rest[1]
    bufs = list(rest[2:2 + RING])
    zbuf = rest[2 + RING]
    i = 3 + RING
    if with_counts:
      cbuf, z16 = rest[i], rest[i + 1]
      i += 2
    else:
      cbuf = z16 = None
    acc = rest[i]
    i += 1
    if with_counts:
      accc = rest[i]
      i += 1
    else:
      accc = None
    sems = rest[i:]
    gsem = [sems[0]] * RING
    ssem = [sems[1]] * RING
    csem = sems[2] if with_counts else None

    c = lax.axis_index("c")
    s = lax.axis_index("s")

    def initbufs(i, _):
      for t in range(HC // 16):
        zbuf[i, pl.ds(t * 16, 16)] = jnp.zeros((16,), jnp.float32)
      if with_counts:
        cbuf[i, :] = jnp.ones((16,), jnp.float32)
        z16[i, :] = jnp.zeros((16,), jnp.float32)
      return 0

    lax.fori_loop(0, B_E, initbufs, 0)

    for k in range(RCH):
      sl = pl.ds(s * ROWS_PT + k * B_E, B_E)
      pltpu.sync_copy(zbuf, acc.at[sl])
      if with_counts:
        pltpu.sync_copy(z16, accc.at[sl])
    plsc.subcore_barrier()

    pltpu.sync_copy(edges_hbm.at[0, pl.ds(s * T_STEPS, T_STEPS)], srcv)
    pltpu.sync_copy(edges_hbm.at[1, pl.ds(s * T_STEPS, T_STEPS)], dstv)

    xc = x_hbm.at[c]
    clo = c * CNT_STEPS

    def group(g, _):
      gds = []
      for b in range(RING):
        j = g * RING + b
        gds.append(pltpu.async_copy(xc.at[srcv.at[j]], bufs[b], gsem[b]))
      for d in gds:
        d.wait()
      sds = []
      for b in range(RING):
        j = g * RING + b
        sds.append(
            pltpu.async_copy(bufs[b], acc.at[dstv.at[j]], ssem[b], add=True))
      if with_counts:
        @pl.when(c == 0)
        def _():
          cds = []
          for b in range(RING):
            j = g * RING + b
            cds.append(
                pltpu.async_copy(cbuf, accc.at[dstv.at[j]], csem, add=True))
          for d in cds:
            d.wait()
      for d in sds:
        d.wait()
      return 0

    lax.fori_loop(0, NGRP, group, 0)
    plsc.subcore_barrier()

    for k in range(RCH):
      sl = pl.ds(s * ROWS_PT + k * B_E, B_E)
      pltpu.sync_copy(acc.at[sl], zbuf)
      pltpu.sync_copy(zbuf, out_sum.at[c, sl])
      if with_counts:
        @pl.when(c == 0)
        def _():
          pltpu.sync_copy(accc.at[sl], cbuf)
          pltpu.sync_copy(cbuf, out_cnt.at[sl])

  out_type = [jax.ShapeDtypeStruct((NC, N_PAD, HC), jnp.float32)]
  scratch = [
      pltpu.VMEM((T_STEPS, B_E), jnp.int32),
      pltpu.VMEM((T_STEPS, B_E), jnp.int32),
  ]
  scratch += [pltpu.VMEM((B_E, HC), jnp.float32)] * RING
  scratch.append(pltpu.VMEM((B_E, HC), jnp.float32))
  if with_counts:
    out_type.append(jax.ShapeDtypeStruct((N_PAD, 16), jnp.float32))
    scratch += [
        pltpu.VMEM((B_E, 16), jnp.float32),
        pltpu.VMEM((B_E, 16), jnp.float32),
    ]
  scratch.append(pltpu.VMEM_SHARED((N_PAD, HC), jnp.float32))
  if with_counts:
    scratch.append(pltpu.VMEM_SHARED((N_PAD, 16), jnp.float32))
  scratch += [pltpu.SemaphoreType.DMA] * (2 + (1 if with_counts else 0))

  return pl.kernel(body, out_type=out_type, mesh=_mesh,
                   scratch_types=scratch, compiler_params=_sc_params)


_sc_agg_cnt = _make_sc_agg(True)
_sc_agg = _make_sc_agg(False)

_BN = 1024


def _dense1_body(s_ref, c_ref, x_ref, wl_ref, wr_ref, b_ref, o_ref):
  cnt = c_ref[:, 0:1]
  mean = jnp.concatenate([s_ref[0], s_ref[1]], axis=1) / jnp.maximum(cnt, 1.0)
  h = (jnp.dot(mean, wl_ref[...], preferred_element_type=jnp.float32)
       + b_ref[...]
       + jnp.dot(x_ref[...], wr_ref[...], preferred_element_type=jnp.float32))
  h = jnp.maximum(h, 0.0)
  o_ref[0] = h[:, :HC]
  o_ref[1] = h[:, HC:]


def _dense2_body(s_ref, c_ref, h_ref, wl_ref, wr_ref, b_ref, ws_ref,
                 o_ref, os_ref):
  cnt = c_ref[:, 0:1]
  mean = jnp.concatenate([s_ref[0], s_ref[1]], axis=1) / jnp.maximum(cnt, 1.0)
  h = jnp.concatenate([h_ref[0], h_ref[1]], axis=1)
  h2 = (jnp.dot(mean, wl_ref[...], preferred_element_type=jnp.float32)
        + b_ref[...]
        + jnp.dot(h, wr_ref[...], preferred_element_type=jnp.float32))
  o_ref[...] = h2
  os_ref[...] = jnp.dot(h2, ws_ref[...], preferred_element_type=jnp.float32)


_split_spec = pl.BlockSpec((NC, _BN, HC), lambda i: (0, i, 0))
_cnt_spec = pl.BlockSpec((_BN, 16), lambda i: (i, 0))
_row_spec = pl.BlockSpec((_BN, D), lambda i: (i, 0))
_w_spec = pl.BlockSpec((D, D), lambda i: (0, 0))
_b_spec = pl.BlockSpec((1, D), lambda i: (0, 0))


def _dense1(s1, cnt, xp, wl, wr, b):
  return pl.pallas_call(
      _dense1_body,
      grid=(N_PAD // _BN,),
      in_specs=[_split_spec, _cnt_spec, _row_spec, _w_spec, _w_spec, _b_spec],
      out_specs=_split_spec,
      out_shape=jax.ShapeDtypeStruct((NC, N_PAD, HC), jnp.float32),
  )(s1, cnt, xp, wl, wr, b)


def _dense2(s2, cnt, h, wl, wr, b, ws):
  ws_spec = pl.BlockSpec((D, D_S), lambda i: (0, 0))
  os_spec = pl.BlockSpec((_BN, D_S), lambda i: (i, 0))
  return pl.pallas_call(
      _dense2_body,
      grid=(N_PAD // _BN,),
      in_specs=[_split_spec, _cnt_spec, _split_spec, _w_spec, _w_spec,
                _b_spec, ws_spec],
      out_specs=[_row_spec, os_spec],
      out_shape=[jax.ShapeDtypeStruct((N_PAD, D), jnp.float32),
                 jax.ShapeDtypeStruct((N_PAD, D_S), jnp.float32)],
  )(s2, cnt, h, wl, wr, b, ws)


@jax.jit
def kernel(x, edge_index, W1_l, b1_l, W1_r, W2_l, b2_l, W2_r, W_s):
  xp = jnp.pad(x, ((0, N_PAD - N), (0, 0)))
  xs = xp.reshape(N_PAD, NC, HC).transpose(1, 0, 2)
  src = jnp.pad(edge_index[0], (0, E_PAD - E))
  dst = jnp.pad(edge_index[1], (0, E_PAD - E), constant_values=N)
  edges = jnp.stack([src, dst]).reshape(2, NS * T_STEPS, B_E)

  s1, cnt = _sc_agg_cnt(xs, edges)
  hs = _dense1(s1, cnt, xp, W1_l.T, W1_r.T, b1_l.reshape(1, D))
  (s2,) = _sc_agg(hs, edges)
  h2, out_s = _dense2(s2, cnt, hs, W2_l.T, W2_r.T, b2_l.reshape(1, D),
                      W_s.T)
  return out_s[:N], h2[:N]


# speedup vs baseline: 4.6180x; 1.0023x over previous
import functools

import jax
import jax.numpy as jnp
from jax import lax
from jax.experimental import pallas as pl
from jax.experimental.pallas import tpu as pltpu
from jax.experimental.pallas import tpu_sc as plsc

N = 10000
D = 128
D_S = 64
E = 320000

NC = 2
NS = 16
HC = D // NC

B_E = 128
KB = 256
E_TILE = 20480
E_PAD = NS * E_TILE
N_PAD = 10240
ROWS_PT = N_PAD // NS
RCH = ROWS_PT // B_E

RING = 2
GCH = RING * KB
NGRP = E_TILE // GCH

_mesh = plsc.VectorSubcoreMesh(
    core_axis_name="c", subcore_axis_name="s", num_cores=NC, num_subcores=NS)
_sc_params = pltpu.CompilerParams(use_tc_tiling_on_sc=False)


def _make_sc_agg(with_counts):
  def body(x_hbm, edges_hbm, *refs):
    if with_counts:
      out_sum, out_cnt = refs[0], refs[1]
      rest = refs[2:]
    else:
      out_sum = refs[0]
      out_cnt = None
      rest = refs[1:]
    isrc, idst = rest[0], rest[1]
    bufs = list(rest[2:2 + RING])
    zbuf = rest[2 + RING]
    i = 3 + RING
    if with_counts:
      cbuf, z16 = rest[i], rest[i + 1]
      i += 2
    else:
      cbuf = z16 = None
    acc = rest[i]
    i += 1
    if with_counts:
      accc = rest[i]
      i += 1
    else:
      accc = None
    sems = rest[i:]
    gsem, ssem, isem = sems[0], sems[1], sems[2]
    csem = sems[3] if with_counts else None

    c = lax.axis_index("c")
    s = lax.axis_index("s")

    def initbufs(i, _):
      for t in range(HC // 16):
        zbuf[i, pl.ds(t * 16, 16)] = jnp.zeros((16,), jnp.float32)
      if with_counts:
        for k in range(KB // B_E):
          cbuf[k * B_E + i, :] = jnp.ones((16,), jnp.float32)
        z16[i, :] = jnp.zeros((16,), jnp.float32)
      return 0

    lax.fori_loop(0, B_E, initbufs, 0)

    for k in range(RCH):
      sl = pl.ds(s * ROWS_PT + k * B_E, B_E)
      pltpu.sync_copy(zbuf, acc.at[sl])
      if with_counts:
        pltpu.sync_copy(z16, accc.at[sl])
    plsc.subcore_barrier()

    xc = x_hbm.at[c]
    esrc = edges_hbm.at[0, s]
    edst = edges_hbm.at[1, s]

    def idx_issue(g, slot):
      pltpu.async_copy(esrc.at[pl.ds(g * GCH, GCH)], isrc.at[slot], isem)
      pltpu.async_copy(edst.at[pl.ds(g * GCH, GCH)], idst.at[slot], isem)

    def idx_wait():
      pltpu.make_async_copy(esrc.at[pl.ds(0, GCH)], isrc.at[0], isem).wait()
      pltpu.make_async_copy(edst.at[pl.ds(0, GCH)], idst.at[0], isem).wait()

    idx_issue(0, 0)

    def group(g, _):
      slot = lax.rem(g, 2)
      idx_wait()

      @pl.when(g + 1 < NGRP)
      def _():
        idx_issue(g + 1, 1 - slot)

      gds = []
      for b in range(RING):
        gds.append(pltpu.async_copy(
            xc.at[isrc.at[slot, pl.ds(b * KB, KB)]], bufs[b], gsem))
      for d in gds:
        d.wait()
      sds = []
      for b in range(RING):
        sds.append(pltpu.async_copy(
            bufs[b], acc.at[idst.at[slot, pl.ds(b * KB, KB)]], ssem,
            add=True))
      if with_counts:
        @pl.when(c == 0)
        def _():
          cds = []
          for b in range(RING):
            cds.append(pltpu.async_copy(
                cbuf, accc.at[idst.at[slot, pl.ds(b * KB, KB)]], csem,
                add=True))
          for d in cds:
            d.wait()
      for d in sds:
        d.wait()
      return 0

    lax.fori_loop(0, NGRP, group, 0)
    plsc.subcore_barrier()

    for k in range(RCH):
      sl = pl.ds(s * ROWS_PT + k * B_E, B_E)
      pltpu.sync_copy(acc.at[sl], zbuf)
      pltpu.sync_copy(zbuf, out_sum.at[c, sl])
      if with_counts:
        @pl.when(c == 0)
        def _():
          pltpu.sync_copy(accc.at[sl], z16)
          pltpu.sync_copy(z16, out_cnt.at[sl])

  out_type = [jax.ShapeDtypeStruct((NC, N_PAD, HC), jnp.float32)]
  scratch = [
      pltpu.VMEM((2, GCH), jnp.int32),
      pltpu.VMEM((2, GCH), jnp.int32),
  ]
  scratch += [pltpu.VMEM((KB, HC), jnp.float32)] * RING
  scratch.append(pltpu.VMEM((B_E, HC), jnp.float32))
  if with_counts:
    out_type.append(jax.ShapeDtypeStruct((N_PAD, 16), jnp.float32))
    scratch += [
        pltpu.VMEM((KB, 16), jnp.float32),
        pltpu.VMEM((B_E, 16), jnp.float32),
    ]
  scratch.append(pltpu.VMEM_SHARED((N_PAD, HC), jnp.float32))
  if with_counts:
    scratch.append(pltpu.VMEM_SHARED((N_PAD, 16), jnp.float32))
  scratch += [pltpu.SemaphoreType.DMA] * (3 + (1 if with_counts else 0))

  return pl.kernel(body, out_type=out_type, mesh=_mesh,
                   scratch_types=scratch, compiler_params=_sc_params)


_sc_agg_cnt = _make_sc_agg(True)
_sc_agg = _make_sc_agg(False)

_BN = 1024


def _dense1_body(s_ref, c_ref, x_ref, wl_ref, wr_ref, b_ref, o_ref):
  cnt = c_ref[:, 0:1]
  mean = jnp.concatenate([s_ref[0], s_ref[1]], axis=1) / jnp.maximum(cnt, 1.0)
  h = (jnp.dot(mean, wl_ref[...], preferred_element_type=jnp.float32)
       + b_ref[...]
       + jnp.dot(x_ref[...], wr_ref[...], preferred_element_type=jnp.float32))
  h = jnp.maximum(h, 0.0)
  o_ref[0] = h[:, :HC]
  o_ref[1] = h[:, HC:]


def _dense2_body(s_ref, c_ref, h_ref, wl_ref, wr_ref, b_ref, ws_ref,
                 o_ref, os_ref):
  cnt = c_ref[:, 0:1]
  mean = jnp.concatenate([s_ref[0], s_ref[1]], axis=1) / jnp.maximum(cnt, 1.0)
  h = jnp.concatenate([h_ref[0], h_ref[1]], axis=1)
  h2 = (jnp.dot(mean, wl_ref[...], preferred_element_type=jnp.float32)
        + b_ref[...]
        + jnp.dot(h, wr_ref[...], preferred_element_type=jnp.float32))
  o_ref[...] = h2
  os_ref[...] = jnp.dot(h2, ws_ref[...], preferred_element_type=jnp.float32)


_split_spec = pl.BlockSpec((NC, _BN, HC), lambda i: (0, i, 0))
_cnt_spec = pl.BlockSpec((_BN, 16), lambda i: (i, 0))
_row_spec = pl.BlockSpec((_BN, D), lambda i: (i, 0))
_w_spec = pl.BlockSpec((D, D), lambda i: (0, 0))
_b_spec = pl.BlockSpec((1, D), lambda i: (0, 0))


def _dense1(s1, cnt, xp, wl, wr, b):
  return pl.pallas_call(
      _dense1_body,
      grid=(N_PAD // _BN,),
      in_specs=[_split_spec, _cnt_spec, _row_spec, _w_spec, _w_spec, _b_spec],
      out_specs=_split_spec,
      out_shape=jax.ShapeDtypeStruct((NC, N_PAD, HC), jnp.float32),
  )(s1, cnt, xp, wl, wr, b)


def _dense2(s2, cnt, h, wl, wr, b, ws):
  ws_spec = pl.BlockSpec((D, D_S), lambda i: (0, 0))
  os_spec = pl.BlockSpec((_BN, D_S), lambda i: (i, 0))
  return pl.pallas_call(
      _dense2_body,
      grid=(N_PAD // _BN,),
      in_specs=[_split_spec, _cnt_spec, _split_spec, _w_spec, _w_spec,
                _b_spec, ws_spec],
      out_specs=[_row_spec, os_spec],
      out_shape=[jax.ShapeDtypeStruct((N_PAD, D), jnp.float32),
                 jax.ShapeDtypeStruct((N_PAD, D_S), jnp.float32)],
  )(s2, cnt, h, wl, wr, b, ws)


@jax.jit
def kernel(x, edge_index, W1_l, b1_l, W1_r, W2_l, b2_l, W2_r, W_s):
  xp = jnp.pad(x, ((0, N_PAD - N), (0, 0)))
  xs = xp.reshape(N_PAD, NC, HC).transpose(1, 0, 2)
  src = jnp.pad(edge_index[0], (0, E_PAD - E))
  dst = jnp.pad(edge_index[1], (0, E_PAD - E), constant_values=N)
  edges = jnp.stack([src, dst]).reshape(2, NS, E_TILE)

  s1, cnt = _sc_agg_cnt(xs, edges)
  hs = _dense1(s1, cnt, xp, W1_l.T, W1_r.T, b1_l.reshape(1, D))
  (s2,) = _sc_agg(hs, edges)
  h2, out_s = _dense2(s2, cnt, hs, W2_l.T, W2_r.T, b2_l.reshape(1, D),
                      W_s.T)
  return out_s[:N], h2[:N]


# speedup vs baseline: 5.1726x; 1.1201x over previous
import functools

import jax
import jax.numpy as jnp
from jax import lax
from jax.experimental import pallas as pl
from jax.experimental.pallas import tpu as pltpu
from jax.experimental.pallas import tpu_sc as plsc

N = 10000
D = 128
D_S = 64
E = 320000

NC = 2
NS = 16
HC = D // NC

B_E = 128
KB = 160
E_TILE = 20480
E_PAD = NS * E_TILE
N_PAD = 10240
ROWS_PT = N_PAD // NS
RCH = ROWS_PT // B_E

SLOTS = 4
DEPTH = 2
TSTEPS = E_TILE // KB
NGRP = TSTEPS // SLOTS
GCH = SLOTS * KB
ISLOTS = 4

_mesh = plsc.VectorSubcoreMesh(
    core_axis_name="c", subcore_axis_name="s", num_cores=NC, num_subcores=NS)
_sc_params = pltpu.CompilerParams(use_tc_tiling_on_sc=False)


def _make_sc_agg(with_counts):
  def body(x_hbm, edges_hbm, *refs):
    if with_counts:
      out_sum, out_cnt = refs[0], refs[1]
      rest = refs[2:]
    else:
      out_sum = refs[0]
      out_cnt = None
      rest = refs[1:]
    isrc, idst = rest[0], rest[1]
    bufs = list(rest[2:2 + SLOTS])
    zbuf = rest[2 + SLOTS]
    i = 3 + SLOTS
    if with_counts:
      cbuf, z16 = rest[i], rest[i + 1]
      i += 2
    else:
      cbuf = z16 = None
    acc = rest[i]
    i += 1
    if with_counts:
      accc = rest[i]
      i += 1
    else:
      accc = None
    sems = rest[i:]
    gsem, ssem = sems[:SLOTS], sems[SLOTS:2 * SLOTS]
    isem = sems[2 * SLOTS]
    csem = sems[2 * SLOTS + 1] if with_counts else None

    c = lax.axis_index("c")
    s = lax.axis_index("s")

    def initbufs(i, _):
      if with_counts:
        cbuf[i, :] = jnp.ones((16,), jnp.float32)

      @pl.when(i < B_E)
      def _():
        for t in range(HC // 16):
          zbuf[i, pl.ds(t * 16, 16)] = jnp.zeros((16,), jnp.float32)
        if with_counts:
          z16[i, :] = jnp.zeros((16,), jnp.float32)
      return 0

    lax.fori_loop(0, max(KB, B_E), initbufs, 0)

    for k in range(RCH):
      sl = pl.ds(s * ROWS_PT + k * B_E, B_E)
      pltpu.sync_copy(zbuf, acc.at[sl])
      if with_counts:
        pltpu.sync_copy(z16, accc.at[sl])
    plsc.subcore_barrier()

    xc = x_hbm.at[c]
    esrc = edges_hbm.at[0, s]
    edst = edges_hbm.at[1, s]

    def idx_issue(g):
      islot = lax.rem(g, ISLOTS)
      pltpu.async_copy(esrc.at[pl.ds(g * GCH, GCH)], isrc.at[islot], isem)
      pltpu.async_copy(edst.at[pl.ds(g * GCH, GCH)], idst.at[islot], isem)

    def idx_wait():
      pltpu.make_async_copy(esrc.at[pl.ds(0, GCH)], isrc.at[0], isem).wait()
      pltpu.make_async_copy(edst.at[pl.ds(0, GCH)], idst.at[0], isem).wait()

    def g_drain(b):
      pltpu.make_async_copy(xc.at[pl.ds(0, KB)], bufs[b], gsem[b]).wait()

    def s_drain(b):
      pltpu.make_async_copy(bufs[b], acc.at[pl.ds(0, KB)], ssem[b]).wait()

    idx_issue(0)
    idx_wait()
    idx_issue(1)
    i0 = lax.rem(jnp.int32(0), ISLOTS)
    for b in range(DEPTH):
      pltpu.async_copy(
          xc.at[isrc.at[i0, pl.ds(b * KB, KB)]], bufs[b], gsem[b])

    def group(g, _):
      ig = lax.rem(g, ISLOTS)
      ig1 = lax.rem(g + 1, ISLOTS)

      @pl.when(g + 1 < NGRP)
      def _():
        idx_wait()

      @pl.when(g + 2 < NGRP)
      def _():
        idx_issue(g + 2)

      for b in range(SLOTS):
        m = g * SLOTS + b
        b2 = (b + DEPTH) % SLOTS
        o2 = ((b + DEPTH) % SLOTS) * KB
        i2 = ig if b < DEPTH else ig1

        if b < DEPTH:
          @pl.when(g > 0)
          def _():
            s_drain(b2)
        else:
          s_drain(b2)

        @pl.when(m + DEPTH < TSTEPS)
        def _():
          pltpu.async_copy(
              xc.at[isrc.at[i2, pl.ds(o2, KB)]], bufs[b2], gsem[b2])

        g_drain(b)
        pltpu.async_copy(
            bufs[b], acc.at[idst.at[ig, pl.ds(b * KB, KB)]], ssem[b],
            add=True)
        if with_counts:
          @pl.when(c == 0)
          def _():
            pltpu.async_copy(
                cbuf, accc.at[idst.at[ig, pl.ds(b * KB, KB)]], csem,
                add=True)

            @pl.when(m >= DEPTH)
            def _():
              pltpu.make_async_copy(
                  cbuf, accc.at[pl.ds(0, KB)], csem).wait()
      return 0

    lax.fori_loop(0, NGRP, group, 0)

    for b in range(SLOTS - DEPTH, SLOTS):
      s_drain(b)
    if with_counts:
      @pl.when(c == 0)
      def _():
        for _i in range(DEPTH):
          pltpu.make_async_copy(cbuf, accc.at[pl.ds(0, KB)], csem).wait()
    plsc.subcore_barrier()

    for k in range(RCH):
      sl = pl.ds(s * ROWS_PT + k * B_E, B_E)
      pltpu.sync_copy(acc.at[sl], zbuf)
      pltpu.sync_copy(zbuf, out_sum.at[c, sl])
      if with_counts:
        @pl.when(c == 0)
        def _():
          pltpu.sync_copy(accc.at[sl], z16)
          pltpu.sync_copy(z16, out_cnt.at[sl])

  out_type = [jax.ShapeDtypeStruct((NC, N_PAD, HC), jnp.float32)]
  scratch = [
      pltpu.VMEM((ISLOTS, GCH), jnp.int32),
      pltpu.VMEM((ISLOTS, GCH), jnp.int32),
  ]
  scratch += [pltpu.VMEM((KB, HC), jnp.float32)] * SLOTS
  scratch.append(pltpu.VMEM((B_E, HC), jnp.float32))
  if with_counts:
    out_type.append(jax.ShapeDtypeStruct((N_PAD, 16), jnp.float32))
    scratch += [
        pltpu.VMEM((KB, 16), jnp.float32),
        pltpu.VMEM((B_E, 16), jnp.float32),
    ]
  scratch.append(pltpu.VMEM_SHARED((N_PAD, HC), jnp.float32))
  if with_counts:
    scratch.append(pltpu.VMEM_SHARED((N_PAD, 16), jnp.float32))
  scratch += [pltpu.SemaphoreType.DMA] * (2 * SLOTS + 1 +
                                          (1 if with_counts else 0))

  return pl.kernel(body, out_type=out_type, mesh=_mesh,
                   scratch_types=scratch, compiler_params=_sc_params)


_sc_agg_cnt = _make_sc_agg(True)
_sc_agg = _make_sc_agg(False)

_BN = 1024


def _dense1_body(s_ref, c_ref, x_ref, wl_ref, wr_ref, b_ref, o_ref):
  cnt = c_ref[:, 0:1]
  mean = jnp.concatenate([s_ref[0], s_ref[1]], axis=1) / jnp.maximum(cnt, 1.0)
  h = (jnp.dot(mean, wl_ref[...], preferred_element_type=jnp.float32)
       + b_ref[...]
       + jnp.dot(x_ref[...], wr_ref[...], preferred_element_type=jnp.float32))
  h = jnp.maximum(h, 0.0)
  o_ref[0] = h[:, :HC]
  o_ref[1] = h[:, HC:]


def _dense2_body(s_ref, c_ref, h_ref, wl_ref, wr_ref, b_ref, ws_ref,
                 o_ref, os_ref):
  cnt = c_ref[:, 0:1]
  mean = jnp.concatenate([s_ref[0], s_ref[1]], axis=1) / jnp.maximum(cnt, 1.0)
  h = jnp.concatenate([h_ref[0], h_ref[1]], axis=1)
  h2 = (jnp.dot(mean, wl_ref[...], preferred_element_type=jnp.float32)
        + b_ref[...]
        + jnp.dot(h, wr_ref[...], preferred_element_type=jnp.float32))
  o_ref[...] = h2
  os_ref[...] = jnp.dot(h2, ws_ref[...], preferred_element_type=jnp.float32)


_split_spec = pl.BlockSpec((NC, _BN, HC), lambda i: (0, i, 0))
_cnt_spec = pl.BlockSpec((_BN, 16), lambda i: (i, 0))
_row_spec = pl.BlockSpec((_BN, D), lambda i: (i, 0))
_w_spec = pl.BlockSpec((D, D), lambda i: (0, 0))
_b_spec = pl.BlockSpec((1, D), lambda i: (0, 0))


def _dense1(s1, cnt, xp, wl, wr, b):
  return pl.pallas_call(
      _dense1_body,
      grid=(N_PAD // _BN,),
      in_specs=[_split_spec, _cnt_spec, _row_spec, _w_spec, _w_spec, _b_spec],
      out_specs=_split_spec,
      out_shape=jax.ShapeDtypeStruct((NC, N_PAD, HC), jnp.float32),
  )(s1, cnt, xp, wl, wr, b)


def _dense2(s2, cnt, h, wl, wr, b, ws):
  ws_spec = pl.BlockSpec((D, D_S), lambda i: (0, 0))
  os_spec = pl.BlockSpec((_BN, D_S), lambda i: (i, 0))
  return pl.pallas_call(
      _dense2_body,
      grid=(N_PAD // _BN,),
      in_specs=[_split_spec, _cnt_spec, _split_spec, _w_spec, _w_spec,
                _b_spec, ws_spec],
      out_specs=[_row_spec, os_spec],
      out_shape=[jax.ShapeDtypeStruct((N_PAD, D), jnp.float32),
                 jax.ShapeDtypeStruct((N_PAD, D_S), jnp.float32)],
  )(s2, cnt, h, wl, wr, b, ws)


@jax.jit
def kernel(x, edge_index, W1_l, b1_l, W1_r, W2_l, b2_l, W2_r, W_s):
  xp = jnp.pad(x, ((0, N_PAD - N), (0, 0)))
  xs = xp.reshape(N_PAD, NC, HC).transpose(1, 0, 2)
  src = jnp.pad(edge_index[0], (0, E_PAD - E))
  dst = jnp.pad(edge_index[1], (0, E_PAD - E), constant_values=N)
  edges = jnp.stack([src, dst]).reshape(2, NS, E_TILE)

  s1, cnt = _sc_agg_cnt(xs, edges)
  hs = _dense1(s1, cnt, xp, W1_l.T, W1_r.T, b1_l.reshape(1, D))
  (s2,) = _sc_agg(hs, edges)
  h2, out_s = _dense2(s2, cnt, hs, W2_l.T, W2_r.T, b2_l.reshape(1, D),
                      W_s.T)
  return out_s[:N], h2[:N]


# speedup vs baseline: 5.1832x; 1.0020x over previous
import functools

import jax
import jax.numpy as jnp
from jax import lax
from jax.experimental import pallas as pl
from jax.experimental.pallas import tpu as pltpu
from jax.experimental.pallas import tpu_sc as plsc

N = 10000
D = 128
D_S = 64
E = 320000

NC = 2
NS = 16
HC = D // NC

B_E = 128
KB = 128
E_TILE = 20480
E_PAD = NS * E_TILE
N_PAD = 10240
ROWS_PT = N_PAD // NS
RCH = ROWS_PT // B_E
ZCH = 64

SLOTS = 8
DEPTH = 4
TSTEPS = E_TILE // KB
NGRP = TSTEPS // SLOTS
GCH = SLOTS * KB
ISLOTS = 3

_mesh = plsc.VectorSubcoreMesh(
    core_axis_name="c", subcore_axis_name="s", num_cores=NC, num_subcores=NS)
_sc_params = pltpu.CompilerParams(use_tc_tiling_on_sc=False)


def _make_sc_agg(with_counts):
  def body(x_hbm, edges_hbm, *refs):
    if with_counts:
      out_sum, out_cnt = refs[0], refs[1]
      rest = refs[2:]
    else:
      out_sum = refs[0]
      out_cnt = None
      rest = refs[1:]
    isrc, idst = rest[0], rest[1]
    bufs = list(rest[2:2 + SLOTS])
    zbuf = bufs[0]
    i = 2 + SLOTS
    if with_counts:
      cbuf, z16 = rest[i], rest[i + 1]
      i += 2
    else:
      cbuf = z16 = None
    acc = rest[i]
    i += 1
    if with_counts:
      accc = rest[i]
      i += 1
    else:
      accc = None
    sems = rest[i:]
    gsem, ssem = sems[:SLOTS], sems[SLOTS:2 * SLOTS]
    isem = sems[2 * SLOTS]
    csem = sems[2 * SLOTS + 1] if with_counts else None

    c = lax.axis_index("c")
    s = lax.axis_index("s")

    def initbufs(i, _):
      if with_counts:
        cbuf[i, :] = jnp.ones((16,), jnp.float32)

      @pl.when(i < B_E)
      def _():
        for t in range(HC // 16):
          zbuf[i, pl.ds(t * 16, 16)] = jnp.zeros((16,), jnp.float32)
      if with_counts:
        @pl.when(i < ZCH)
        def _():
          z16[i, :] = jnp.zeros((16,), jnp.float32)
      return 0

    lax.fori_loop(0, max(KB, B_E), initbufs, 0)

    for k in range(RCH):
      pltpu.sync_copy(zbuf, acc.at[pl.ds(s * ROWS_PT + k * B_E, B_E)])
    if with_counts:
      for k in range(ROWS_PT // ZCH):
        pltpu.sync_copy(z16, accc.at[pl.ds(s * ROWS_PT + k * ZCH, ZCH)])
    plsc.subcore_barrier()

    xc = x_hbm.at[c]
    esrc = edges_hbm.at[0, s]
    edst = edges_hbm.at[1, s]

    def idx_issue(g):
      islot = lax.rem(g, ISLOTS)
      pltpu.async_copy(esrc.at[pl.ds(g * GCH, GCH)], isrc.at[islot], isem)
      pltpu.async_copy(edst.at[pl.ds(g * GCH, GCH)], idst.at[islot], isem)

    def idx_wait():
      pltpu.make_async_copy(esrc.at[pl.ds(0, GCH)], isrc.at[0], isem).wait()
      pltpu.make_async_copy(edst.at[pl.ds(0, GCH)], idst.at[0], isem).wait()

    def g_drain(b):
      pltpu.make_async_copy(xc.at[pl.ds(0, KB)], bufs[b], gsem[b]).wait()

    def s_drain(b):
      pltpu.make_async_copy(bufs[b], acc.at[pl.ds(0, KB)], ssem[b]).wait()

    idx_issue(0)
    idx_wait()
    idx_issue(1)
    i0 = lax.rem(jnp.int32(0), ISLOTS)
    for b in range(DEPTH):
      pltpu.async_copy(
          xc.at[isrc.at[i0, pl.ds(b * KB, KB)]], bufs[b], gsem[b])

    def group(g, _):
      ig = lax.rem(g, ISLOTS)
      ig1 = lax.rem(g + 1, ISLOTS)

      @pl.when(g + 1 < NGRP)
      def _():
        idx_wait()

      for b in range(SLOTS):
        if b == DEPTH:
          @pl.when(g + 2 < NGRP)
          def _():
            idx_issue(g + 2)
        m = g * SLOTS + b
        b2 = (b + DEPTH) % SLOTS
        o2 = ((b + DEPTH) % SLOTS) * KB
        i2 = ig if b < DEPTH else ig1

        if b < DEPTH:
          @pl.when(g > 0)
          def _():
            s_drain(b2)
        else:
          s_drain(b2)

        @pl.when(m + DEPTH < TSTEPS)
        def _():
          pltpu.async_copy(
              xc.at[isrc.at[i2, pl.ds(o2, KB)]], bufs[b2], gsem[b2])

        g_drain(b)
        pltpu.async_copy(
            bufs[b], acc.at[idst.at[ig, pl.ds(b * KB, KB)]], ssem[b],
            add=True)
        if with_counts:
          @pl.when(c == 0)
          def _():
            pltpu.async_copy(
                cbuf, accc.at[idst.at[ig, pl.ds(b * KB, KB)]], csem,
                add=True)

            @pl.when(m >= DEPTH)
            def _():
              pltpu.make_async_copy(
                  cbuf, accc.at[pl.ds(0, KB)], csem).wait()
      return 0

    lax.fori_loop(0, NGRP, group, 0)

    for b in range(SLOTS - DEPTH, SLOTS):
      s_drain(b)
    if with_counts:
      @pl.when(c == 0)
      def _():
        for _i in range(DEPTH):
          pltpu.make_async_copy(cbuf, accc.at[pl.ds(0, KB)], csem).wait()
    plsc.subcore_barrier()

    for k in range(RCH):
      sl = pl.ds(s * ROWS_PT + k * B_E, B_E)
      pltpu.sync_copy(acc.at[sl], zbuf)
      pltpu.sync_copy(zbuf, out_sum.at[c, sl])
      if with_counts:
        @pl.when(c == 0)
        def _():
          for q in range(B_E // ZCH):
            zl = pl.ds(s * ROWS_PT + k * B_E + q * ZCH, ZCH)
            pltpu.sync_copy(accc.at[zl], z16)
            pltpu.sync_copy(z16, out_cnt.at[zl])

  out_type = [jax.ShapeDtypeStruct((NC, N_PAD, HC), jnp.float32)]
  scratch = [
      pltpu.VMEM((ISLOTS, GCH), jnp.int32),
      pltpu.VMEM((ISLOTS, GCH), jnp.int32),
  ]
  scratch += [pltpu.VMEM((KB, HC), jnp.float32)] * SLOTS
  if with_counts:
    out_type.append(jax.ShapeDtypeStruct((N_PAD, 16), jnp.float32))
    scratch += [
        pltpu.VMEM((KB, 16), jnp.float32),
        pltpu.VMEM((ZCH, 16), jnp.float32),
    ]
  scratch.append(pltpu.VMEM_SHARED((N_PAD, HC), jnp.float32))
  if with_counts:
    scratch.append(pltpu.VMEM_SHARED((N_PAD, 16), jnp.float32))
  scratch += [pltpu.SemaphoreType.DMA] * (2 * SLOTS + 1 +
                                          (1 if with_counts else 0))

  return pl.kernel(body, out_type=out_type, mesh=_mesh,
                   scratch_types=scratch, compiler_params=_sc_params)


_sc_agg_cnt = _make_sc_agg(True)
_sc_agg = _make_sc_agg(False)

_BN = 1024


def _dense1_body(s_ref, c_ref, x_ref, wl_ref, wr_ref, b_ref, o_ref):
  cnt = c_ref[:, 0:1]
  mean = jnp.concatenate([s_ref[0], s_ref[1]], axis=1) / jnp.maximum(cnt, 1.0)
  h = (jnp.dot(mean, wl_ref[...], preferred_element_type=jnp.float32)
       + b_ref[...]
       + jnp.dot(x_ref[...], wr_ref[...], preferred_element_type=jnp.float32))
  h = jnp.maximum(h, 0.0)
  o_ref[0] = h[:, :HC]
  o_ref[1] = h[:, HC:]


def _dense2_body(s_ref, c_ref, h_ref, wl_ref, wr_ref, b_ref, ws_ref,
                 o_ref, os_ref):
  cnt = c_ref[:, 0:1]
  mean = jnp.concatenate([s_ref[0], s_ref[1]], axis=1) / jnp.maximum(cnt, 1.0)
  h = jnp.concatenate([h_ref[0], h_ref[1]], axis=1)
  h2 = (jnp.dot(mean, wl_ref[...], preferred_element_type=jnp.float32)
        + b_ref[...]
        + jnp.dot(h, wr_ref[...], preferred_element_type=jnp.float32))
  o_ref[...] = h2
  os_ref[...] = jnp.dot(h2, ws_ref[...], preferred_element_type=jnp.float32)


_split_spec = pl.BlockSpec((NC, _BN, HC), lambda i: (0, i, 0))
_cnt_spec = pl.BlockSpec((_BN, 16), lambda i: (i, 0))
_row_spec = pl.BlockSpec((_BN, D), lambda i: (i, 0))
_w_spec = pl.BlockSpec((D, D), lambda i: (0, 0))
_b_spec = pl.BlockSpec((1, D), lambda i: (0, 0))


def _dense1(s1, cnt, xp, wl, wr, b):
  return pl.pallas_call(
      _dense1_body,
      grid=(N_PAD // _BN,),
      in_specs=[_split_spec, _cnt_spec, _row_spec, _w_spec, _w_spec, _b_spec],
      out_specs=_split_spec,
      out_shape=jax.ShapeDtypeStruct((NC, N_PAD, HC), jnp.float32),
  )(s1, cnt, xp, wl, wr, b)


def _dense2(s2, cnt, h, wl, wr, b, ws):
  ws_spec = pl.BlockSpec((D, D_S), lambda i: (0, 0))
  os_spec = pl.BlockSpec((_BN, D_S), lambda i: (i, 0))
  return pl.pallas_call(
      _dense2_body,
      grid=(N_PAD // _BN,),
      in_specs=[_split_spec, _cnt_spec, _split_spec, _w_spec, _w_spec,
                _b_spec, ws_spec],
      out_specs=[_row_spec, os_spec],
      out_shape=[jax.ShapeDtypeStruct((N_PAD, D), jnp.float32),
                 jax.ShapeDtypeStruct((N_PAD, D_S), jnp.float32)],
  )(s2, cnt, h, wl, wr, b, ws)


@jax.jit
def kernel(x, edge_index, W1_l, b1_l, W1_r, W2_l, b2_l, W2_r, W_s):
  xp = jnp.pad(x, ((0, N_PAD - N), (0, 0)))
  xs = xp.reshape(N_PAD, NC, HC).transpose(1, 0, 2)
  src = jnp.pad(edge_index[0], (0, E_PAD - E))
  dst = jnp.pad(edge_index[1], (0, E_PAD - E), constant_values=N)
  edges = jnp.stack([src, dst]).reshape(2, NS, E_TILE)

  s1, cnt = _sc_agg_cnt(xs, edges)
  hs = _dense1(s1, cnt, xp, W1_l.T, W1_r.T, b1_l.reshape(1, D))
  (s2,) = _sc_agg(hs, edges)
  h2, out_s = _dense2(s2, cnt, hs, W2_l.T, W2_r.T, b2_l.reshape(1, D),
                      W_s.T)
  return out_s[:N], h2[:N]


# speedup vs baseline: 5.4744x; 1.0562x over previous
import functools

import jax
import jax.numpy as jnp
from jax import lax
from jax.experimental import pallas as pl
from jax.experimental.pallas import tpu as pltpu
from jax.experimental.pallas import tpu_sc as plsc

N = 10000
D = 128
D_S = 64
E = 320000

NC = 2
NS = 16
HC = D // NC

B_E = 128
KB = 128
E_TILE = 20480
E_PAD = NS * E_TILE
N_PAD = 10240
ROWS_PT = N_PAD // NS
RCH = ROWS_PT // B_E
ZCH = 64

SLOTS = 8
DEPTH = 4
TSTEPS = E_TILE // KB
NGRP = TSTEPS // SLOTS
GCH = SLOTS * KB
ISLOTS = 3

_mesh = plsc.VectorSubcoreMesh(
    core_axis_name="c", subcore_axis_name="s", num_cores=NC, num_subcores=NS)
_sc_params = pltpu.CompilerParams(use_tc_tiling_on_sc=False)


def _make_sc_agg(with_counts):
  def body(x_hbm, edges_hbm, *refs):
    if with_counts:
      out_sum, out_cnt = refs[0], refs[1]
      rest = refs[2:]
    else:
      out_sum = refs[0]
      out_cnt = None
      rest = refs[1:]
    isrc, idst = rest[0], rest[1]
    bufs = list(rest[2:2 + SLOTS])
    zbuf = bufs[0]
    i = 2 + SLOTS
    if with_counts:
      cbuf, z16 = rest[i], rest[i + 1]
      i += 2
    else:
      cbuf = z16 = None
    acc = rest[i]
    i += 1
    if with_counts:
      accc = rest[i]
      i += 1
    else:
      accc = None
    sems = rest[i:]
    gsem, ssem = sems[:SLOTS], sems[SLOTS:2 * SLOTS]
    isem = sems[2 * SLOTS]
    csem = sems[2 * SLOTS + 1] if with_counts else None

    c = lax.axis_index("c")
    s = lax.axis_index("s")

    def initbufs(i, _):
      if with_counts:
        cbuf[i, :] = jnp.ones((16,), jnp.float32)

      @pl.when(i < B_E)
      def _():
        for t in range(HC // 16):
          zbuf[i, pl.ds(t * 16, 16)] = jnp.zeros((16,), jnp.float32)
      if with_counts:
        @pl.when(i < ZCH)
        def _():
          z16[i, :] = jnp.zeros((16,), jnp.float32)
      return 0

    lax.fori_loop(0, max(KB, B_E), initbufs, 0)

    for k in range(RCH):
      pltpu.sync_copy(zbuf, acc.at[pl.ds(s * ROWS_PT + k * B_E, B_E)])
    if with_counts:
      for k in range(ROWS_PT // ZCH):
        pltpu.sync_copy(z16, accc.at[pl.ds(s * ROWS_PT + k * ZCH, ZCH)])
    plsc.subcore_barrier()

    xc = x_hbm.at[c]
    esrc = edges_hbm.at[0, s]
    edst = edges_hbm.at[1, s]

    def idx_issue(g):
      islot = lax.rem(g, ISLOTS)
      pltpu.async_copy(esrc.at[pl.ds(g * GCH, GCH)], isrc.at[islot], isem)
      pltpu.async_copy(edst.at[pl.ds(g * GCH, GCH)], idst.at[islot], isem)

    def idx_wait():
      pltpu.make_async_copy(esrc.at[pl.ds(0, GCH)], isrc.at[0], isem).wait()
      pltpu.make_async_copy(edst.at[pl.ds(0, GCH)], idst.at[0], isem).wait()

    def g_drain(b):
      pltpu.make_async_copy(xc.at[pl.ds(0, KB)], bufs[b], gsem[b]).wait()

    def s_drain(b):
      pltpu.make_async_copy(bufs[b], acc.at[pl.ds(0, KB)], ssem[b]).wait()

    idx_issue(0)
    idx_wait()
    idx_issue(1)
    i0 = lax.rem(jnp.int32(0), ISLOTS)
    for b in range(DEPTH):
      pltpu.async_copy(
          xc.at[isrc.at[i0, pl.ds(b * KB, KB)]], bufs[b], gsem[b])

    def group(g, _):
      ig = lax.rem(g, ISLOTS)
      ig1 = lax.rem(g + 1, ISLOTS)

      @pl.when(g + 1 < NGRP)
      def _():
        idx_wait()

      for b in range(SLOTS):
        if b == DEPTH:
          @pl.when(g + 2 < NGRP)
          def _():
            idx_issue(g + 2)
        m = g * SLOTS + b
        b2 = (b + DEPTH) % SLOTS
        o2 = ((b + DEPTH) % SLOTS) * KB
        i2 = ig if b < DEPTH else ig1

        if b < DEPTH:
          @pl.when(g > 0)
          def _():
            s_drain(b2)
        else:
          s_drain(b2)

        @pl.when(m + DEPTH < TSTEPS)
        def _():
          pltpu.async_copy(
              xc.at[isrc.at[i2, pl.ds(o2, KB)]], bufs[b2], gsem[b2])

        g_drain(b)
        pltpu.async_copy(
            bufs[b], acc.at[idst.at[ig, pl.ds(b * KB, KB)]], ssem[b],
            add=True)
        if with_counts:
          @pl.when(c == 0)
          def _():
            pltpu.async_copy(
                cbuf, accc.at[idst.at[ig, pl.ds(b * KB, KB)]], csem,
                add=True)

            @pl.when(m >= DEPTH)
            def _():
              pltpu.make_async_copy(
                  cbuf, accc.at[pl.ds(0, KB)], csem).wait()
      return 0

    lax.fori_loop(0, NGRP, group, 0)

    for b in range(SLOTS - DEPTH, SLOTS):
      s_drain(b)
    if with_counts:
      @pl.when(c == 0)
      def _():
        for _i in range(DEPTH):
          pltpu.make_async_copy(cbuf, accc.at[pl.ds(0, KB)], csem).wait()
    plsc.subcore_barrier()

    for k in range(RCH):
      sl = pl.ds(s * ROWS_PT + k * B_E, B_E)
      pltpu.sync_copy(acc.at[sl], zbuf)
      pltpu.sync_copy(zbuf, out_sum.at[c, sl])
      if with_counts:
        @pl.when(c == 0)
        def _():
          for q in range(B_E // ZCH):
            zl = pl.ds(s * ROWS_PT + k * B_E + q * ZCH, ZCH)
            pltpu.sync_copy(accc.at[zl], z16)
            pltpu.sync_copy(z16, out_cnt.at[zl])

  out_type = [jax.ShapeDtypeStruct((NC, N_PAD, HC), jnp.float32)]
  scratch = [
      pltpu.VMEM((ISLOTS, GCH), jnp.int32),
      pltpu.VMEM((ISLOTS, GCH), jnp.int32),
  ]
  scratch += [pltpu.VMEM((KB, HC), jnp.float32)] * SLOTS
  if with_counts:
    out_type.append(jax.ShapeDtypeStruct((N_PAD, 16), jnp.float32))
    scratch += [
        pltpu.VMEM((KB, 16), jnp.float32),
        pltpu.VMEM((ZCH, 16), jnp.float32),
    ]
  scratch.append(pltpu.VMEM_SHARED((N_PAD, HC), jnp.float32))
  if with_counts:
    scratch.append(pltpu.VMEM_SHARED((N_PAD, 16), jnp.float32))
  scratch += [pltpu.SemaphoreType.DMA] * (2 * SLOTS + 1 +
                                          (1 if with_counts else 0))

  return pl.kernel(body, out_type=out_type, mesh=_mesh,
                   scratch_types=scratch, compiler_params=_sc_params)


_sc_agg_cnt = _make_sc_agg(True)
_sc_agg = _make_sc_agg(False)

_BN = 1024


def _xr1_body(x_ref, wr_ref, b_ref, o_ref):
  o_ref[...] = (jnp.dot(x_ref[...], wr_ref[...],
                        preferred_element_type=jnp.float32) + b_ref[...])


def _xr2_body(h_ref, wr_ref, b_ref, o_ref):
  h = jnp.concatenate([h_ref[0], h_ref[1]], axis=1)
  o_ref[...] = (jnp.dot(h, wr_ref[...],
                        preferred_element_type=jnp.float32) + b_ref[...])


def _dense1_body(s_ref, c_ref, xr_ref, wl_ref, o_ref):
  cnt = c_ref[:, 0:1]
  mean = jnp.concatenate([s_ref[0], s_ref[1]], axis=1) / jnp.maximum(cnt, 1.0)
  h = (jnp.dot(mean, wl_ref[...], preferred_element_type=jnp.float32)
       + xr_ref[...])
  h = jnp.maximum(h, 0.0)
  o_ref[0] = h[:, :HC]
  o_ref[1] = h[:, HC:]


def _dense2_body(s_ref, c_ref, hr_ref, wl_ref, ws_ref, o_ref, os_ref):
  cnt = c_ref[:, 0:1]
  mean = jnp.concatenate([s_ref[0], s_ref[1]], axis=1) / jnp.maximum(cnt, 1.0)
  h2 = (jnp.dot(mean, wl_ref[...], preferred_element_type=jnp.float32)
        + hr_ref[...])
  o_ref[...] = h2
  os_ref[...] = jnp.dot(h2, ws_ref[...], preferred_element_type=jnp.float32)


_split_spec = pl.BlockSpec((NC, _BN, HC), lambda i: (0, i, 0))
_cnt_spec = pl.BlockSpec((_BN, 16), lambda i: (i, 0))
_row_spec = pl.BlockSpec((_BN, D), lambda i: (i, 0))
_w_spec = pl.BlockSpec((D, D), lambda i: (0, 0))
_b_spec = pl.BlockSpec((1, D), lambda i: (0, 0))


def _xr1(xp, wr, b):
  return pl.pallas_call(
      _xr1_body,
      grid=(N_PAD // _BN,),
      in_specs=[_row_spec, _w_spec, _b_spec],
      out_specs=_row_spec,
      out_shape=jax.ShapeDtypeStruct((N_PAD, D), jnp.float32),
  )(xp, wr, b)


def _xr2(hs, wr, b):
  return pl.pallas_call(
      _xr2_body,
      grid=(N_PAD // _BN,),
      in_specs=[_split_spec, _w_spec, _b_spec],
      out_specs=_row_spec,
      out_shape=jax.ShapeDtypeStruct((N_PAD, D), jnp.float32),
  )(hs, wr, b)


def _dense1(s1, cnt, xr, wl):
  return pl.pallas_call(
      _dense1_body,
      grid=(N_PAD // _BN,),
      in_specs=[_split_spec, _cnt_spec, _row_spec, _w_spec],
      out_specs=_split_spec,
      out_shape=jax.ShapeDtypeStruct((NC, N_PAD, HC), jnp.float32),
  )(s1, cnt, xr, wl)


def _dense2(s2, cnt, hr, wl, ws):
  ws_spec = pl.BlockSpec((D, D_S), lambda i: (0, 0))
  os_spec = pl.BlockSpec((_BN, D_S), lambda i: (i, 0))
  return pl.pallas_call(
      _dense2_body,
      grid=(N_PAD // _BN,),
      in_specs=[_split_spec, _cnt_spec, _row_spec, _w_spec, ws_spec],
      out_specs=[_row_spec, os_spec],
      out_shape=[jax.ShapeDtypeStruct((N_PAD, D), jnp.float32),
                 jax.ShapeDtypeStruct((N_PAD, D_S), jnp.float32)],
  )(s2, cnt, hr, wl, ws)


@jax.jit
def kernel(x, edge_index, W1_l, b1_l, W1_r, W2_l, b2_l, W2_r, W_s):
  xp = jnp.pad(x, ((0, N_PAD - N), (0, 0)))
  xs = xp.reshape(N_PAD, NC, HC).transpose(1, 0, 2)
  src = jnp.pad(edge_index[0], (0, E_PAD - E))
  dst = jnp.pad(edge_index[1], (0, E_PAD - E), constant_values=N)
  edges = jnp.stack([src, dst]).reshape(2, NS, E_TILE)

  xr = _xr1(xp, W1_r.T, b1_l.reshape(1, D))
  s1, cnt = _sc_agg_cnt(xs, edges)
  hs = _dense1(s1, cnt, xr, W1_l.T)
  hr = _xr2(hs, W2_r.T, b2_l.reshape(1, D))
  (s2,) = _sc_agg(hs, edges)
  h2, out_s = _dense2(s2, cnt, hr, W2_l.T, W_s.T)
  return out_s[:N], h2[:N]


# speedup vs baseline: 5.4775x; 1.0006x over previous
import jax
import jax.numpy as jnp
from jax import lax
from jax.experimental import pallas as pl
from jax.experimental.pallas import tpu as pltpu
from jax.experimental.pallas import tpu_sc as plsc

N = 10000
D = 128
D_S = 64
E = 320000

NC = 2
NS = 16
HC = D // NC

B_E = 128
KB = 128
E_TILE = 20480
E_PAD = NS * E_TILE
N_PAD = 10240
ROWS_PT = N_PAD // NS
RCH = ROWS_PT // B_E
ZCH = 64

SLOTS = 8
DEPTH = 4
TSTEPS = E_TILE // KB
NGRP = TSTEPS // SLOTS
GCH = SLOTS * KB
ISLOTS = 3

_mesh = plsc.VectorSubcoreMesh(
    core_axis_name="c", subcore_axis_name="s", num_cores=NC, num_subcores=NS)
_sc_params = pltpu.CompilerParams(use_tc_tiling_on_sc=False)


def _make_sc_agg(with_counts):
  def body(x_hbm, edges_hbm, *refs):
    if with_counts:
      out_sum, out_cnt = refs[0], refs[1]
      rest = refs[2:]
    else:
      out_sum = refs[0]
      out_cnt = None
      rest = refs[1:]
    isrc, idst = rest[0], rest[1]
    bufs = list(rest[2:2 + SLOTS])
    zbuf = bufs[0]
    i = 2 + SLOTS
    if with_counts:
      cbuf, z16 = rest[i], rest[i + 1]
      i += 2
    else:
      cbuf = z16 = None
    acc = rest[i]
    i += 1
    if with_counts:
      accc = rest[i]
      i += 1
    else:
      accc = None
    sems = rest[i:]
    gsem, ssem = sems[:SLOTS], sems[SLOTS:2 * SLOTS]
    isem = sems[2 * SLOTS]
    csem = sems[2 * SLOTS + 1] if with_counts else None

    c = lax.axis_index("c")
    s = lax.axis_index("s")

    def initbufs(i, _):
      if with_counts:
        cbuf[i, :] = jnp.ones((16,), jnp.float32)

      @pl.when(i < B_E)
      def _():
        for t in range(HC // 16):
          zbuf[i, pl.ds(t * 16, 16)] = jnp.zeros((16,), jnp.float32)
      if with_counts:
        @pl.when(i < ZCH)
        def _():
          z16[i, :] = jnp.zeros((16,), jnp.float32)
      return 0

    lax.fori_loop(0, max(KB, B_E), initbufs, 0)

    for k in range(RCH):
      pltpu.sync_copy(zbuf, acc.at[pl.ds(s * ROWS_PT + k * B_E, B_E)])
    if with_counts:
      for k in range(ROWS_PT // ZCH):
        pltpu.sync_copy(z16, accc.at[pl.ds(s * ROWS_PT + k * ZCH, ZCH)])
    plsc.subcore_barrier()

    xc = x_hbm.at[c]
    esrc = edges_hbm.at[0, s]
    edst = edges_hbm.at[1, s]

    def idx_issue(g):
      islot = lax.rem(g, ISLOTS)
      pltpu.async_copy(esrc.at[pl.ds(g * GCH, GCH)], isrc.at[islot], isem)
      pltpu.async_copy(edst.at[pl.ds(g * GCH, GCH)], idst.at[islot], isem)

    def idx_wait():
      pltpu.make_async_copy(esrc.at[pl.ds(0, GCH)], isrc.at[0], isem).wait()
      pltpu.make_async_copy(edst.at[pl.ds(0, GCH)], idst.at[0], isem).wait()

    def g_drain(b):
      pltpu.make_async_copy(xc.at[pl.ds(0, KB)], bufs[b], gsem[b]).wait()

    def s_drain(b):
      pltpu.make_async_copy(bufs[b], acc.at[pl.ds(0, KB)], ssem[b]).wait()

    idx_issue(0)
    idx_wait()
    idx_issue(1)
    i0 = lax.rem(jnp.int32(0), ISLOTS)
    for b in range(DEPTH):
      pltpu.async_copy(
          xc.at[isrc.at[i0, pl.ds(b * KB, KB)]], bufs[b], gsem[b])

    def group(g, _):
      ig = lax.rem(g, ISLOTS)
      ig1 = lax.rem(g + 1, ISLOTS)

      @pl.when(g + 1 < NGRP)
      def _():
        idx_wait()

      for b in range(SLOTS):
        if b == DEPTH:
          @pl.when(g + 2 < NGRP)
          def _():
            idx_issue(g + 2)
        m = g * SLOTS + b
        b2 = (b + DEPTH) % SLOTS
        o2 = ((b + DEPTH) % SLOTS) * KB
        i2 = ig if b < DEPTH else ig1

        if b < DEPTH:
          @pl.when(g > 0)
          def _():
            s_drain(b2)
        else:
          s_drain(b2)

        @pl.when(m + DEPTH < TSTEPS)
        def _():
          pltpu.async_copy(
              xc.at[isrc.at[i2, pl.ds(o2, KB)]], bufs[b2], gsem[b2])

        g_drain(b)
        pltpu.async_copy(
            bufs[b], acc.at[idst.at[ig, pl.ds(b * KB, KB)]], ssem[b],
            add=True)
        if with_counts:
          @pl.when(c == 0)
          def _():
            pltpu.async_copy(
                cbuf, accc.at[idst.at[ig, pl.ds(b * KB, KB)]], csem,
                add=True)

            @pl.when(m >= DEPTH)
            def _():
              pltpu.make_async_copy(
                  cbuf, accc.at[pl.ds(0, KB)], csem).wait()
      return 0

    lax.fori_loop(0, NGRP, group, 0)

    for b in range(SLOTS - DEPTH, SLOTS):
      s_drain(b)
    if with_counts:
      @pl.when(c == 0)
      def _():
        for _i in range(DEPTH):
          pltpu.make_async_copy(cbuf, accc.at[pl.ds(0, KB)], csem).wait()
    plsc.subcore_barrier()

    for k in range(RCH):
      sl = pl.ds(s * ROWS_PT + k * B_E, B_E)
      pltpu.sync_copy(acc.at[sl], zbuf)
      pltpu.sync_copy(zbuf, out_sum.at[c, sl])
      if with_counts:
        @pl.when(c == 0)
        def _():
          for q in range(B_E // ZCH):
            zl = pl.ds(s * ROWS_PT + k * B_E + q * ZCH, ZCH)
            pltpu.sync_copy(accc.at[zl], z16)
            pltpu.sync_copy(z16, out_cnt.at[zl])

  out_type = [jax.ShapeDtypeStruct((NC, N_PAD, HC), jnp.float32)]
  scratch = [
      pltpu.VMEM((ISLOTS, GCH), jnp.int32),
      pltpu.VMEM((ISLOTS, GCH), jnp.int32),
  ]
  scratch += [pltpu.VMEM((KB, HC), jnp.float32)] * SLOTS
  if with_counts:
    out_type.append(jax.ShapeDtypeStruct((N_PAD, 16), jnp.float32))
    scratch += [
        pltpu.VMEM((KB, 16), jnp.float32),
        pltpu.VMEM((ZCH, 16), jnp.float32),
    ]
  scratch.append(pltpu.VMEM_SHARED((N_PAD, HC), jnp.float32))
  if with_counts:
    scratch.append(pltpu.VMEM_SHARED((N_PAD, 16), jnp.float32))
  scratch += [pltpu.SemaphoreType.DMA] * (2 * SLOTS + 1 +
                                          (1 if with_counts else 0))

  return pl.kernel(body, out_type=out_type, mesh=_mesh,
                   scratch_types=scratch, compiler_params=_sc_params)


_sc_agg_cnt = _make_sc_agg(True)
_sc_agg = _make_sc_agg(False)

_BN = 1024


def _xr1_body(x_ref, wr_ref, b_ref, o_ref):
  o_ref[...] = (jnp.dot(x_ref[...], wr_ref[...],
                        preferred_element_type=jnp.float32) + b_ref[...])


def _xr2_body(h_ref, wr_ref, b_ref, o_ref):
  h = jnp.concatenate([h_ref[0], h_ref[1]], axis=1)
  o_ref[...] = (jnp.dot(h, wr_ref[...],
                        preferred_element_type=jnp.float32) + b_ref[...])


def _dense1_body(s_ref, c_ref, xr_ref, wl_ref, o_ref):
  cnt = c_ref[:, 0:1]
  mean = jnp.concatenate([s_ref[0], s_ref[1]], axis=1) / jnp.maximum(cnt, 1.0)
  h = (jnp.dot(mean, wl_ref[...], preferred_element_type=jnp.float32)
       + xr_ref[...])
  h = jnp.maximum(h, 0.0)
  o_ref[0] = h[:, :HC]
  o_ref[1] = h[:, HC:]


def _dense2_body(s_ref, c_ref, hr_ref, wl_ref, ws_ref, o_ref, os_ref):
  cnt = c_ref[:, 0:1]
  mean = jnp.concatenate([s_ref[0], s_ref[1]], axis=1) / jnp.maximum(cnt, 1.0)
  h2 = (jnp.dot(mean, wl_ref[...], preferred_element_type=jnp.float32)
        + hr_ref[...])
  o_ref[...] = h2
  os_ref[...] = jnp.dot(h2, ws_ref[...], preferred_element_type=jnp.float32)


_split_spec = pl.BlockSpec((NC, _BN, HC), lambda i: (0, i, 0))
_cnt_spec = pl.BlockSpec((_BN, 16), lambda i: (i, 0))
_row_spec = pl.BlockSpec((_BN, D), lambda i: (i, 0))
_w_spec = pl.BlockSpec((D, D), lambda i: (0, 0))
_b_spec = pl.BlockSpec((1, D), lambda i: (0, 0))


def _xr1(xp, wr, b):
  return pl.pallas_call(
      _xr1_body,
      grid=(N_PAD // _BN,),
      in_specs=[_row_spec, _w_spec, _b_spec],
      out_specs=_row_spec,
      out_shape=jax.ShapeDtypeStruct((N_PAD, D), jnp.float32),
  )(xp, wr, b)


def _xr2(hs, wr, b):
  return pl.pallas_call(
      _xr2_body,
      grid=(N_PAD // _BN,),
      in_specs=[_split_spec, _w_spec, _b_spec],
      out_specs=_row_spec,
      out_shape=jax.ShapeDtypeStruct((N_PAD, D), jnp.float32),
  )(hs, wr, b)


def _dense1(s1, cnt, xr, wl):
  return pl.pallas_call(
      _dense1_body,
      grid=(N_PAD // _BN,),
      in_specs=[_split_spec, _cnt_spec, _row_spec, _w_spec],
      out_specs=_split_spec,
      out_shape=jax.ShapeDtypeStruct((NC, N_PAD, HC), jnp.float32),
  )(s1, cnt, xr, wl)


def _dense2(s2, cnt, hr, wl, ws):
  ws_spec = pl.BlockSpec((D, D_S), lambda i: (0, 0))
  os_spec = pl.BlockSpec((_BN, D_S), lambda i: (i, 0))
  return pl.pallas_call(
      _dense2_body,
      grid=(N_PAD // _BN,),
      in_specs=[_split_spec, _cnt_spec, _row_spec, _w_spec, ws_spec],
      out_specs=[_row_spec, os_spec],
      out_shape=[jax.ShapeDtypeStruct((N_PAD, D), jnp.float32),
                 jax.ShapeDtypeStruct((N_PAD, D_S), jnp.float32)],
  )(s2, cnt, hr, wl, ws)


@jax.jit
def kernel(x, edge_index, W1_l, b1_l, W1_r, W2_l, b2_l, W2_r, W_s):
  xp = jnp.pad(x, ((0, N_PAD - N), (0, 0)))
  xs = xp.reshape(N_PAD, NC, HC).transpose(1, 0, 2)
  src = jnp.pad(edge_index[0], (0, E_PAD - E))
  dst = jnp.pad(edge_index[1], (0, E_PAD - E), constant_values=N)
  edges = jnp.stack([src, dst]).reshape(2, NS, E_TILE)

  xr = _xr1(xp, W1_r.T, b1_l.reshape(1, D))
  s1, cnt = _sc_agg_cnt(xs, edges)
  hs = _dense1(s1, cnt, xr, W1_l.T)
  hr = _xr2(hs, W2_r.T, b2_l.reshape(1, D))
  (s2,) = _sc_agg(hs, edges)
  h2, out_s = _dense2(s2, cnt, hr, W2_l.T, W_s.T)
  return out_s[:N], h2[:N]
